# Initial kernel scaffold; baseline (speedup 1.0000x reference)
import jax
import jax.numpy as jnp
from jax.experimental import pallas as pl


def kernel(x, W0, W1, W2, W3, W4):
    raise NotImplementedError("write your pallas kernel here")


# speedup vs baseline: 3.0050x; 3.0050x over previous
import functools

import jax
import jax.numpy as jnp
from jax import lax
from jax.experimental import pallas as pl
from jax.experimental.pallas import tpu as pltpu
from jax.experimental.pallas import tpu_sc as plsc

_SEQ = 200
_LAG = 50
_D = 64
_B = 1024
_NC = 2
_NS = 16
_NW = _NC * _NS
_BPW = _B // _NW
_ICH = 100
_NCH = _SEQ // _ICH


def _body(xidx_hbm, w0, w1, w2, w3, w4, cidx_hbm, out_hbm,
          xidx_v, cidx_v, rows0, rows1, c2, c3, c4, sem):
  cid = lax.axis_index("c")
  sid = lax.axis_index("s")
  wid = sid * _NC + cid

  pltpu.sync_copy(w2, c2)
  pltpu.sync_copy(cidx_hbm, cidx_v)
  for k in range(_NCH):
    pltpu.async_copy(w3.at[cidx_v.at[0, k]],
                     c3.at[pl.ds(k * _ICH, _ICH)], sem)
    pltpu.async_copy(w4.at[cidx_v.at[1, k]],
                     c4.at[pl.ds(k * _ICH, _ICH)], sem)
  for k in range(_NCH):
    pltpu.make_async_copy(w3.at[cidx_v.at[0, k]],
                          c3.at[pl.ds(k * _ICH, _ICH)], sem).wait()
    pltpu.make_async_copy(w4.at[cidx_v.at[1, k]],
                          c4.at[pl.ds(k * _ICH, _ICH)], sem).wait()

  def body(i, carry):
    b = wid * _BPW + i
    pltpu.sync_copy(xidx_hbm.at[b], xidx_v)
    for k in range(_NCH):
      pltpu.async_copy(w0.at[xidx_v.at[0, k]],
                       rows0.at[pl.ds(k * _ICH, _ICH)], sem)
      pltpu.async_copy(w1.at[xidx_v.at[1, k]],
                       rows1.at[pl.ds(k * _ICH, _ICH)], sem)
    for k in range(_NCH):
      pltpu.make_async_copy(w0.at[xidx_v.at[0, k]],
                            rows0.at[pl.ds(k * _ICH, _ICH)], sem).wait()
      pltpu.make_async_copy(w1.at[xidx_v.at[1, k]],
                            rows1.at[pl.ds(k * _ICH, _ICH)], sem).wait()
    pltpu.sync_copy(rows0, out_hbm.at[b, :, 0, :])
    pltpu.sync_copy(rows1, out_hbm.at[b, :, 1, :])
    pltpu.sync_copy(c2, out_hbm.at[b, :, 2, :])
    pltpu.sync_copy(c3, out_hbm.at[b, :, 3, :])
    pltpu.sync_copy(c4, out_hbm.at[b, :, 4, :])
    return carry

  lax.fori_loop(0, _BPW, body, 0)


def kernel(x, W0, W1, W2, W3, W4):
  xidx = x.astype(jnp.int32).transpose(0, 2, 1).reshape(_B, 2, _NCH, _ICH)
  pf = jnp.concatenate([jnp.zeros(_SEQ - _LAG, jnp.int32),
                        jnp.arange(1, _LAG + 1, dtype=jnp.int32)])
  isf = (jnp.arange(_SEQ, dtype=jnp.int32) >= (_SEQ - _LAG)).astype(jnp.int32)
  cidx = jnp.stack([pf, isf]).reshape(2, _NCH, _ICH)

  mesh = plsc.VectorSubcoreMesh(core_axis_name="c", subcore_axis_name="s")
  run = pl.kernel(
      _body,
      out_type=jax.ShapeDtypeStruct((_B, _SEQ, 5, _D), jnp.float32),
      mesh=mesh,
      scratch_types=[
          pltpu.VMEM((2, _NCH, _ICH), jnp.int32),
          pltpu.VMEM((2, _NCH, _ICH), jnp.int32),
          pltpu.VMEM((_SEQ, _D), jnp.float32),
          pltpu.VMEM((_SEQ, _D), jnp.float32),
          pltpu.VMEM((_SEQ, _D), jnp.float32),
          pltpu.VMEM((_SEQ, _D), jnp.float32),
          pltpu.VMEM((_SEQ, _D), jnp.float32),
          pltpu.SemaphoreType.DMA,
      ],
      compiler_params=pltpu.CompilerParams(use_tc_tiling_on_sc=False),
  )
  return run(xidx, W0, W1, W2, W3, W4, cidx)
